# baseline (speedup 1.0000x reference)
import functools

import jax
import jax.numpy as jnp
from jax import lax
from jax.experimental import pallas as pl
from jax.experimental.pallas import tpu as pltpu
from jax.experimental.pallas import tpu_sc as plsc

_R = 5
_N = 4096
_L = 4096
_NC = 2
_NS = 16
_NW = _NC * _NS
_NSC = 3072
_NTC = _N - _NSC
_ROWS = _NSC // _NW
_LANES = 16

_NEGINF = float("-inf")
_POSINF = float("inf")
_GRP = 8
_K = 8
_NB = 2
_NG = _ROWS // _K
_BT = 128


def _perm(v, idx):
    return lax.gather(
        v, idx[:, None],
        lax.GatherDimensionNumbers(offset_dims=(), collapsed_slice_dims=(0,),
                                   start_index_map=(0,)),
        slice_sizes=(1,), mode=lax.GatherScatterMode.PROMISE_IN_BOUNDS)


def _row_result(buf, r, leff):
    iota = lax.iota(jnp.int32, _LANES)

    def _bin(j, vec):
        s = (j * leff) // _R
        e = ((j + 1) * leff + (_R - 1)) // _R
        c0 = (s // _LANES) * _LANES
        v0 = buf[r, pl.ds(pl.multiple_of(c0, _LANES), _LANES)]
        m0 = (iota >= s - c0) & (iota < e - c0)
        am0 = jnp.where(m0, v0, _NEGINF)
        an0 = jnp.where(m0, v0, _POSINF)
        n_int = jnp.maximum((e - c0) // _LANES - 1, 0)
        c1 = c0 + _LANES
        n_grp = n_int // _GRP

        def _group(base, am_, an_):
            vs = [buf[r, pl.ds(pl.multiple_of(base + u * _LANES, _LANES),
                               _LANES)] for u in range(_GRP)]
            mxs, mns = list(vs), list(vs)
            while len(mxs) > 1:
                mxs = [jnp.maximum(a, b) for a, b in zip(mxs[::2], mxs[1::2])]
                mns = [jnp.minimum(a, b) for a, b in zip(mns[::2], mns[1::2])]
            return jnp.maximum(am_, mxs[0]), jnp.minimum(an_, mns[0])

        @plsc.parallel_loop(0, n_grp, carry=(am0, an0))
        def _interior(g, carry, c1=c1):
            return _group(c1 + g * (_GRP * _LANES), *carry)

        am, an = _interior
        base_o = c1 + jnp.maximum(n_int - _GRP, 0) * _LANES
        am_o, an_o = _group(base_o, am, an)
        big = n_int >= _GRP
        am = jnp.where(big, am_o, am)
        an = jnp.where(big, an_o, an)

        @plsc.parallel_loop(0, jnp.where(big, 0, n_int), carry=(am, an))
        def _singles(t, carry, c1=c1):
            am_, an_ = carry
            v = buf[r, pl.ds(pl.multiple_of(c1 + t * _LANES, _LANES),
                             _LANES)]
            return jnp.maximum(am_, v), jnp.minimum(an_, v)

        am, an = _singles
        pt = c1 + n_int * _LANES
        vt = buf[r, pl.ds(pl.multiple_of(jnp.minimum(pt, _L - _LANES),
                                         _LANES), _LANES)]
        mt = iota < (e - pt)
        am = jnp.maximum(am, jnp.where(mt, vt, _NEGINF))
        an = jnp.minimum(an, jnp.where(mt, vt, _POSINF))
        for sh in (1, 2, 4, 8):
            am = jnp.maximum(am, _perm(am, iota ^ sh))
            an = jnp.minimum(an, _perm(an, iota ^ sh))
        vec = jnp.where(iota == j, an, vec)
        vec = jnp.where(iota == (_R + j), am, vec)
        return vec

    vec = lax.fori_loop(0, _R, _bin,
                        jnp.full((_LANES,), _POSINF, jnp.float32))
    for k in (2, 4, 8, 16):
        sh = k // 2
        while sh >= 1:
            p = _perm(vec, iota ^ sh)
            want_min = ((iota & sh) == 0) != ((iota & k) != 0)
            vec = jnp.where(want_min, jnp.minimum(vec, p),
                            jnp.maximum(vec, p))
            sh //= 2
    return vec


def _sc_body(x_hbm, len_hbm, out_hbm, len_v, buf0, buf1, outv, sem0, sem1):
    wid = lax.axis_index("s") * _NC + lax.axis_index("c")
    base = wid * _ROWS
    pltpu.sync_copy(len_hbm.at[pl.ds(base * _LANES, _ROWS * _LANES)], len_v)
    bufs = (buf0, buf1)
    sems = (sem0, sem1)
    for b in range(_NB - 1):
        pltpu.async_copy(x_hbm.at[pl.ds(base + b * _K, _K)], bufs[b], sems[b])

    def outer(gq, _):
        for kb in range(_NB):
            g = gq * _NB + kb
            nk = (kb + _NB - 1) % _NB

            @pl.when(g + _NB - 1 < _NG)
            def _():
                pltpu.async_copy(
                    x_hbm.at[pl.ds(base + (g + _NB - 1) * _K, _K)],
                    bufs[nk], sems[nk])

            pltpu.make_async_copy(x_hbm.at[pl.ds(base + g * _K, _K)],
                                  bufs[kb], sems[kb]).wait()

            def row_loop(rr, _2, kb=kb, g=g):
                i = g * _K + rr
                lv = len_v[pl.ds(pl.multiple_of(i * _LANES, _LANES), _LANES)]
                leff = lv[0]
                outv[i, :] = _row_result(bufs[kb], rr, leff)
                return 0

            lax.fori_loop(0, _K, row_loop, 0)
        return 0

    lax.fori_loop(0, _NG // _NB, outer, 0)
    pltpu.sync_copy(outv, out_hbm.at[pl.ds(base, _ROWS)])


def _minmax16(inputs, lengths16):
    mesh = plsc.VectorSubcoreMesh(core_axis_name="c", subcore_axis_name="s")
    f = functools.partial(
        pl.kernel,
        out_type=jax.ShapeDtypeStruct((_NSC, _LANES), jnp.float32),
        mesh=mesh,
        scratch_types=[
            pltpu.VMEM((_ROWS * _LANES,), jnp.int32),
            pltpu.VMEM((_K, _L), jnp.float32),
            pltpu.VMEM((_K, _L), jnp.float32),
            pltpu.VMEM((_ROWS, _LANES), jnp.float32),
            pltpu.SemaphoreType.DMA,
            pltpu.SemaphoreType.DMA,
        ],
    )(_sc_body)
    return f(inputs, lengths16)


def _swap_blocks(v, sh):
    chunks = [v[i * sh:(i + 1) * sh] for i in range(16 // sh)]
    out = []
    for i in range(0, len(chunks), 2):
        out.extend([chunks[i + 1], chunks[i]])
    return jnp.concatenate(out, axis=0)


def _tc_body(se_ref, x_ref, o_ref):
    x = x_ref[...]
    pos = lax.broadcasted_iota(jnp.int32, (_BT, _L), 1)
    rows = []
    for j in range(_R):
        s = se_ref[:, j:j + 1]
        e = se_ref[:, _R + j:_R + j + 1]
        m = (pos >= s) & (pos < e)
        mn = jnp.min(jnp.where(m, x, _POSINF), axis=1)
        mx = jnp.max(jnp.where(m, x, _NEGINF), axis=1)
        rows.append(mn[None, :])
        rows.append(mx[None, :])
    pad = jnp.full((6, _BT), _POSINF, jnp.float32)
    v = jnp.concatenate(rows + [pad], axis=0)
    riota = lax.broadcasted_iota(jnp.int32, (16, 1), 0)
    for k in (2, 4, 8, 16):
        sh = k // 2
        while sh >= 1:
            p = _swap_blocks(v, sh)
            wm = ((riota & sh) == 0) != ((riota & k) != 0)
            v = jnp.where(wm, jnp.minimum(v, p), jnp.maximum(v, p))
            sh //= 2
    o_ref[...] = v


def _tc_minmax(inputs, se):
    row0 = _NSC // _BT
    return pl.pallas_call(
        _tc_body,
        grid=(_NTC // _BT,),
        in_specs=[
            pl.BlockSpec((_BT, 16), lambda i: (row0 + i, 0)),
            pl.BlockSpec((_BT, _L), lambda i: (row0 + i, 0)),
        ],
        out_specs=pl.BlockSpec((16, _BT), lambda i: (0, i)),
        out_shape=jax.ShapeDtypeStruct((16, _NTC), jnp.float32),
    )(se, inputs)


@jax.jit
def _minmax_all(inputs, lengths16, se):
    out_sc = _minmax16(inputs, lengths16)
    out_tc = _tc_minmax(inputs, se)
    return out_sc, out_tc


def kernel(inputs, lengths):
    leff = jnp.clip(lengths.astype(jnp.int32), 1, _L)
    lengths16 = jnp.repeat(leff[:_NSC], _LANES)
    js = jnp.arange(_R)
    starts = (js[None, :] * leff[:, None]) // _R
    ends = ((js[None, :] + 1) * leff[:, None] + (_R - 1)) // _R
    se = jnp.concatenate([starts, ends, jnp.zeros((_N, 6), jnp.int32)],
                         axis=1)
    out_sc, out_tc = _minmax_all(inputs, lengths16, se)
    return jnp.concatenate([out_sc[:, : 2 * _R], out_tc.T[:, : 2 * _R]],
                           axis=0)


# speedup vs baseline: 1.2193x; 1.2193x over previous
import functools

import jax
import jax.numpy as jnp
from jax import lax
from jax.experimental import pallas as pl
from jax.experimental.pallas import tpu as pltpu
from jax.experimental.pallas import tpu_sc as plsc

_R = 5
_N = 4096
_L = 4096
_NC = 2
_NS = 16
_NW = _NC * _NS
_NSC = 2304
_NTC = _N - _NSC
_ROWS = _NSC // _NW
_LANES = 16

_NEGINF = float("-inf")
_POSINF = float("inf")
_GRP = 8
_K = 4
_NB = 2
_NG = _ROWS // _K
_BT = 128


def _perm(v, idx):
    return lax.gather(
        v, idx[:, None],
        lax.GatherDimensionNumbers(offset_dims=(), collapsed_slice_dims=(0,),
                                   start_index_map=(0,)),
        slice_sizes=(1,), mode=lax.GatherScatterMode.PROMISE_IN_BOUNDS)


def _row_result(buf, r, leff):
    iota = lax.iota(jnp.int32, _LANES)

    def _bin(j, vec):
        s = (j * leff) // _R
        e = ((j + 1) * leff + (_R - 1)) // _R
        c0 = (s // _LANES) * _LANES
        v0 = buf[r, pl.ds(pl.multiple_of(c0, _LANES), _LANES)]
        m0 = (iota >= s - c0) & (iota < e - c0)
        am0 = jnp.where(m0, v0, _NEGINF)
        an0 = jnp.where(m0, v0, _POSINF)
        n_int = jnp.maximum((e - c0) // _LANES - 1, 0)
        c1 = c0 + _LANES
        n_grp = n_int // _GRP

        def _group(base, am_, an_):
            vs = [buf[r, pl.ds(pl.multiple_of(base + u * _LANES, _LANES),
                               _LANES)] for u in range(_GRP)]
            mxs, mns = list(vs), list(vs)
            while len(mxs) > 1:
                mxs = [jnp.maximum(a, b) for a, b in zip(mxs[::2], mxs[1::2])]
                mns = [jnp.minimum(a, b) for a, b in zip(mns[::2], mns[1::2])]
            return jnp.maximum(am_, mxs[0]), jnp.minimum(an_, mns[0])

        @plsc.parallel_loop(0, n_grp, carry=(am0, an0))
        def _interior(g, carry, c1=c1):
            return _group(c1 + g * (_GRP * _LANES), *carry)

        am, an = _interior
        base_o = c1 + jnp.maximum(n_int - _GRP, 0) * _LANES
        am_o, an_o = _group(base_o, am, an)
        big = n_int >= _GRP
        am = jnp.where(big, am_o, am)
        an = jnp.where(big, an_o, an)

        @plsc.parallel_loop(0, jnp.where(big, 0, n_int), carry=(am, an))
        def _singles(t, carry, c1=c1):
            am_, an_ = carry
            v = buf[r, pl.ds(pl.multiple_of(c1 + t * _LANES, _LANES),
                             _LANES)]
            return jnp.maximum(am_, v), jnp.minimum(an_, v)

        am, an = _singles
        pt = c1 + n_int * _LANES
        vt = buf[r, pl.ds(pl.multiple_of(jnp.minimum(pt, _L - _LANES),
                                         _LANES), _LANES)]
        mt = iota < (e - pt)
        am = jnp.maximum(am, jnp.where(mt, vt, _NEGINF))
        an = jnp.minimum(an, jnp.where(mt, vt, _POSINF))
        for sh in (1, 2, 4, 8):
            am = jnp.maximum(am, _perm(am, iota ^ sh))
            an = jnp.minimum(an, _perm(an, iota ^ sh))
        vec = jnp.where(iota == j, an, vec)
        vec = jnp.where(iota == (_R + j), am, vec)
        return vec

    vec = lax.fori_loop(0, _R, _bin,
                        jnp.full((_LANES,), _POSINF, jnp.float32))
    for k in (2, 4, 8, 16):
        sh = k // 2
        while sh >= 1:
            p = _perm(vec, iota ^ sh)
            want_min = ((iota & sh) == 0) != ((iota & k) != 0)
            vec = jnp.where(want_min, jnp.minimum(vec, p),
                            jnp.maximum(vec, p))
            sh //= 2
    return vec


def _sc_body(x_hbm, len_hbm, out_hbm, len_v, buf0, buf1, outv, sem0, sem1):
    wid = lax.axis_index("s") * _NC + lax.axis_index("c")
    base = wid * _ROWS
    pltpu.sync_copy(len_hbm.at[pl.ds(base * _LANES, _ROWS * _LANES)], len_v)
    bufs = (buf0, buf1)
    sems = (sem0, sem1)
    for b in range(_NB - 1):
        pltpu.async_copy(x_hbm.at[pl.ds(base + b * _K, _K)], bufs[b], sems[b])

    def outer(gq, _):
        for kb in range(_NB):
            g = gq * _NB + kb
            nk = (kb + _NB - 1) % _NB

            @pl.when(g + _NB - 1 < _NG)
            def _():
                pltpu.async_copy(
                    x_hbm.at[pl.ds(base + (g + _NB - 1) * _K, _K)],
                    bufs[nk], sems[nk])

            pltpu.make_async_copy(x_hbm.at[pl.ds(base + g * _K, _K)],
                                  bufs[kb], sems[kb]).wait()

            def row_loop(rr, _2, kb=kb, g=g):
                i = g * _K + rr
                lv = len_v[pl.ds(pl.multiple_of(i * _LANES, _LANES), _LANES)]
                leff = lv[0]
                outv[i, :] = _row_result(bufs[kb], rr, leff)
                return 0

            lax.fori_loop(0, _K, row_loop, 0)
        return 0

    lax.fori_loop(0, _NG // _NB, outer, 0)
    pltpu.sync_copy(outv, out_hbm.at[pl.ds(base, _ROWS)])


def _minmax16(inputs, lengths16):
    mesh = plsc.VectorSubcoreMesh(core_axis_name="c", subcore_axis_name="s")
    f = functools.partial(
        pl.kernel,
        out_type=jax.ShapeDtypeStruct((_NSC, _LANES), jnp.float32),
        mesh=mesh,
        scratch_types=[
            pltpu.VMEM((_ROWS * _LANES,), jnp.int32),
            pltpu.VMEM((_K, _L), jnp.float32),
            pltpu.VMEM((_K, _L), jnp.float32),
            pltpu.VMEM((_ROWS, _LANES), jnp.float32),
            pltpu.SemaphoreType.DMA,
            pltpu.SemaphoreType.DMA,
        ],
    )(_sc_body)
    return f(inputs, lengths16)


def _swap_blocks(v, sh):
    chunks = [v[i * sh:(i + 1) * sh] for i in range(16 // sh)]
    out = []
    for i in range(0, len(chunks), 2):
        out.extend([chunks[i + 1], chunks[i]])
    return jnp.concatenate(out, axis=0)


def _tc_body(se_ref, x_ref, o_ref):
    x = x_ref[...]
    pos = lax.broadcasted_iota(jnp.int32, (_BT, _L), 1)
    rows = []
    for j in range(_R):
        s = se_ref[:, j:j + 1]
        e = se_ref[:, _R + j:_R + j + 1]
        m = (pos >= s) & (pos < e)
        mn = jnp.min(jnp.where(m, x, _POSINF), axis=1)
        mx = jnp.max(jnp.where(m, x, _NEGINF), axis=1)
        rows.append(mn[None, :])
        rows.append(mx[None, :])
    pad = jnp.full((6, _BT), _POSINF, jnp.float32)
    v = jnp.concatenate(rows + [pad], axis=0)
    riota = lax.broadcasted_iota(jnp.int32, (16, 1), 0)
    for k in (2, 4, 8, 16):
        sh = k // 2
        while sh >= 1:
            p = _swap_blocks(v, sh)
            wm = ((riota & sh) == 0) != ((riota & k) != 0)
            v = jnp.where(wm, jnp.minimum(v, p), jnp.maximum(v, p))
            sh //= 2
    o_ref[...] = v


def _tc_minmax(inputs, se):
    row0 = _NSC // _BT
    return pl.pallas_call(
        _tc_body,
        grid=(_NTC // _BT,),
        in_specs=[
            pl.BlockSpec((_BT, 16), lambda i: (row0 + i, 0)),
            pl.BlockSpec((_BT, _L), lambda i: (row0 + i, 0)),
        ],
        out_specs=pl.BlockSpec((16, _BT), lambda i: (0, i)),
        out_shape=jax.ShapeDtypeStruct((16, _NTC), jnp.float32),
    )(se, inputs)


@jax.jit
def _minmax_all(inputs, lengths16, se):
    out_sc = _minmax16(inputs, lengths16)
    out_tc = _tc_minmax(inputs, se)
    return out_sc, out_tc


def kernel(inputs, lengths):
    leff = jnp.clip(lengths.astype(jnp.int32), 1, _L)
    lengths16 = jnp.repeat(leff[:_NSC], _LANES)
    js = jnp.arange(_R)
    starts = (js[None, :] * leff[:, None]) // _R
    ends = ((js[None, :] + 1) * leff[:, None] + (_R - 1)) // _R
    se = jnp.concatenate([starts, ends, jnp.zeros((_N, 6), jnp.int32)],
                         axis=1)
    out_sc, out_tc = _minmax_all(inputs, lengths16, se)
    return jnp.concatenate([out_sc[:, : 2 * _R], out_tc.T[:, : 2 * _R]],
                           axis=0)
